# baseline (speedup 1.0000x reference)
import jax
import jax.numpy as jnp
from jax import lax
from jax.experimental import pallas as pl
from jax.experimental.pallas import tpu as pltpu
from jax.experimental.pallas import tpu_sc as plsc

NC, NS, LANES = 2, 16, 16
NW = NC * NS
NSLOT = 4

INPUT_BITS = 4096
N_IN = 1024
N_STATE = 512
N_OUT = 512
NBITS = N_IN + N_STATE
K = 14
KP = 16
MEM = 1 << K
B = 1024
W = B // 2
NG = B // 32

_FMT = plsc.PackFormat.INTERLEAVED

_MESH = plsc.VectorSubcoreMesh(
    core_axis_name="c", subcore_axis_name="s", num_cores=NC, num_subcores=NS
)


def _make_layer(mode):
    R = 32 if mode == "input" else 16
    out_rows = N_OUT if mode == "out" else NBITS
    out_width = B if mode == "out" else W
    out_dtype = jnp.float32 if mode == "out" else jnp.int32

    def body(src, conn, mem, out, conn_l, rows_, tab_, orow_, zbuf, sem_, osem_):
        wid = lax.axis_index("s") * NC + lax.axis_index("c")

        pltpu.sync_copy(conn.at[pl.ds(wid * R, R)], conn_l)

        def start(j, s):
            neuron = j * NW + wid
            pltpu.async_copy(src.at[conn_l.at[j]], rows_[s], sem_[s])
            pltpu.async_copy(mem.at[neuron], tab_[s], sem_[s])

        def wait_slot(s):
            pltpu.make_async_copy(src.at[pl.ds(0, KP)], rows_[s], sem_[s]).wait()
            pltpu.make_async_copy(mem.at[0], tab_[s], sem_[s]).wait()

        def compute(j, s):
            rows16, tab, orow = rows_[s], tab_[s], orow_[s]
            neuron = j * NW + wid
            out_row = neuron + (N_IN if mode == "state" else 0)
            for g in range(NG):
                sl = pl.ds(g * 16, 16)
                r = [plsc.bitcast(rows16[k, sl], jnp.bfloat16)
                     for k in range(K)]
                lo = r[0]
                hi = r[7]
                for k in range(1, 7):
                    ck = jnp.bfloat16(1 << k)
                    lo = lo + r[k] * ck
                    hi = hi + r[k + 7] * ck
                lo_p, lo_q = plsc.unpack(lo, format=_FMT,
                                         preferred_element_type=jnp.float32)
                hi_p, hi_q = plsc.unpack(hi, format=_FMT,
                                         preferred_element_type=jnp.float32)
                bits = []
                for off, lo_h, hi_h in ((0, lo_p, hi_p), (16, lo_q, hi_q)):
                    a = (lo_h + hi_h * 128.0).astype(jnp.int32) & (MEM - 1)
                    v = plsc.load_gather(tab, [a])
                    if mode == "out":
                        orow[pl.ds(g * 32 + off, 16)] = v
                    else:
                        bits.append(jnp.where(v > 0.5, jnp.float32(1.0),
                                              jnp.float32(0.0)))
                if mode != "out":
                    orow[sl] = plsc.bitcast(
                        plsc.pack(bits[0], bits[1], format=_FMT), jnp.int32)
            pltpu.async_copy(orow, out.at[out_row], osem_[s])

        def drain_out(s):
            pltpu.make_async_copy(orow_[s], out.at[0], osem_[s]).wait()

        if mode == "input":
            for r in range(16):
                for g in range(W // 16):
                    zbuf[r, pl.ds(g * 16, 16)] = jnp.zeros((16,), jnp.int32)
            pltpu.sync_copy(zbuf, out.at[pl.ds(N_IN + wid * 16, 16)])
        elif mode == "state":
            for p in range(2):
                sl = pl.ds(wid * 32 + p * 16, 16)
                pltpu.sync_copy(src.at[sl], zbuf)
                pltpu.sync_copy(zbuf, out.at[sl])

        for i in range(NSLOT):
            start(i, i)

        @pl.loop(0, R, step=NSLOT)
        def _(j):
            for s in range(NSLOT):
                wait_slot(s)

                @pl.when(j + s >= NSLOT)
                def _():
                    drain_out(s)

                compute(j + s, s)

                @pl.when(j + s + NSLOT < R)
                def _():
                    start(j + s + NSLOT, s)

        for s in range(NSLOT):
            drain_out(s)

    def wrapped(src, conn, mem, out, conn_l, r0, r1, r2, r3, t0, t1, t2, t3,
                o0, o1, o2, o3, zbuf, s0, s1, s2, s3, os0, os1, os2, os3):
        body(src, conn, mem, out, conn_l, (r0, r1, r2, r3), (t0, t1, t2, t3),
             (o0, o1, o2, o3), zbuf, (s0, s1, s2, s3), (os0, os1, os2, os3))

    return pl.kernel(
        wrapped,
        out_type=jax.ShapeDtypeStruct((out_rows, out_width), out_dtype),
        mesh=_MESH,
        compiler_params=pltpu.CompilerParams(needs_layout_passes=False, use_tc_tiling_on_sc=False),
        scratch_types=(
            [pltpu.VMEM((R, KP), jnp.int32)]
            + [pltpu.VMEM((KP, W), jnp.int32)] * NSLOT
            + [pltpu.VMEM((MEM,), jnp.float32)] * NSLOT
            + [pltpu.VMEM((out_width,), out_dtype)] * NSLOT
            + [pltpu.VMEM((16, W), jnp.int32)]
            + [pltpu.SemaphoreType.DMA] * (2 * NSLOT)
        ),
        name=f"ram_layer_{mode}",
    )


def _pad_conn(conn, r):
    n, k = conn.shape
    p = jnp.concatenate(
        [conn.astype(jnp.int32), jnp.zeros((n, KP - k), jnp.int32)], axis=1)
    return p.reshape(r, NW, KP).swapaxes(0, 1).reshape(n, KP)


@jax.jit
def kernel(x, conn_in, conn_state, conn_out, mem_in, mem_state, mem_out):
    xT = jax.lax.bitcast_convert_type(
        x.astype(jnp.bfloat16).T.reshape(INPUT_BITS, W, 2), jnp.int32)
    conn_in_p = _pad_conn(conn_in, 32)
    conn_state_p = _pad_conn(conn_state, 16)
    conn_out_p = _pad_conn(conn_out, 16)

    layer_in = _make_layer("input")
    layer_state = _make_layer("state")
    layer_out = _make_layer("out")

    bits = layer_in(xT, conn_in_p, mem_in)
    for _ in range(4):
        bits = layer_state(bits, conn_state_p, mem_state)
    out_t = layer_out(bits, conn_out_p, mem_out)
    out_t = out_t.reshape(N_OUT, NG, 2, 16).transpose(0, 1, 3, 2)
    return out_t.reshape(N_OUT, B).T


# speedup vs baseline: 1.1987x; 1.1987x over previous
import jax
import jax.numpy as jnp
from jax import lax
from jax.experimental import pallas as pl
from jax.experimental.pallas import tpu as pltpu
from jax.experimental.pallas import tpu_sc as plsc

NC, NS, LANES = 2, 16, 16
NW = NC * NS
NSLOT = 4

INPUT_BITS = 4096
N_IN = 1024
N_STATE = 512
N_OUT = 512
NBITS = N_IN + N_STATE
K = 14
KP = 16
MEM = 1 << K
B = 1024
W = B // 2
NG = B // 32

_FMT = plsc.PackFormat.INTERLEAVED

_MESH = plsc.VectorSubcoreMesh(
    core_axis_name="c", subcore_axis_name="s", num_cores=NC, num_subcores=NS
)


def _make_layer(mode):
    R = 32 if mode == "input" else 16
    out_rows = N_OUT if mode == "out" else NBITS
    out_width = B if mode == "out" else W
    out_dtype = jnp.float32 if mode == "out" else jnp.int32

    def body(src, conn, mem, out, conn_l, rows_, tab_, orow_, zbuf,
             sem_, osem_):
        wid = lax.axis_index("s") * NC + lax.axis_index("c")

        pltpu.sync_copy(conn.at[pl.ds(wid * R, R)], conn_l)

        def start(j, s):
            neuron = j * NW + wid
            pltpu.async_copy(src.at[conn_l.at[j, pl.ds(0, 8)]],
                             rows_[s].at[pl.ds(0, 8)], sem_[s])
            pltpu.async_copy(src.at[conn_l.at[j, pl.ds(8, 8)]],
                             rows_[s].at[pl.ds(8, 8)], sem_[s])
            pltpu.async_copy(mem.at[neuron], tab_[s], sem_[s])

        def wait_slot(s):
            pltpu.make_async_copy(src.at[pl.ds(0, KP)], rows_[s], sem_[s]).wait()
            pltpu.make_async_copy(mem.at[0], tab_[s], sem_[s]).wait()

        def compute(j, s):
            rows16, tab, orow = rows_[s], tab_[s], orow_[s]
            neuron = j * NW + wid
            out_row = neuron + (N_IN if mode == "state" else 0)
            for g in range(NG):
                sl = pl.ds(g * 16, 16)
                r = [plsc.bitcast(rows16[k, sl], jnp.bfloat16)
                     for k in range(K)]
                lo = r[0]
                hi = r[7]
                for k in range(1, 7):
                    ck = jnp.bfloat16(1 << k)
                    lo = lo + r[k] * ck
                    hi = hi + r[k + 7] * ck
                lo_p, lo_q = plsc.unpack(lo, format=_FMT,
                                         preferred_element_type=jnp.float32)
                hi_p, hi_q = plsc.unpack(hi, format=_FMT,
                                         preferred_element_type=jnp.float32)
                bits = []
                for off, lo_h, hi_h in ((0, lo_p, hi_p), (16, lo_q, hi_q)):
                    a = (lo_h + hi_h * 128.0).astype(jnp.int32) & (MEM - 1)
                    v = plsc.load_gather(tab, [a])
                    if mode == "out":
                        orow[pl.ds(g * 32 + off, 16)] = v
                    else:
                        bits.append(jnp.where(v > 0.5, jnp.float32(1.0),
                                              jnp.float32(0.0)))
                if mode != "out":
                    orow[sl] = plsc.bitcast(
                        plsc.pack(bits[0], bits[1], format=_FMT), jnp.int32)
            pltpu.async_copy(orow, out.at[out_row], osem_[s])

        def drain_out(s):
            pltpu.make_async_copy(orow_[s], out.at[0], osem_[s]).wait()

        if mode == "input":
            for r in range(16):
                for g in range(W // 16):
                    zbuf[r, pl.ds(g * 16, 16)] = jnp.zeros((16,), jnp.int32)
            pltpu.sync_copy(zbuf, out.at[pl.ds(N_IN + wid * 16, 16)])
        elif mode == "state":
            for p in range(2):
                sl = pl.ds(wid * 32 + p * 16, 16)
                pltpu.sync_copy(src.at[sl], zbuf)
                pltpu.sync_copy(zbuf, out.at[sl])

        for i in range(NSLOT):
            start(i, i)

        @pl.loop(0, R, step=NSLOT)
        def _(j):
            for s in range(NSLOT):
                wait_slot(s)

                @pl.when(j + s >= NSLOT)
                def _():
                    drain_out(s)

                compute(j + s, s)

                @pl.when(j + s + NSLOT < R)
                def _():
                    start(j + s + NSLOT, s)

        for s in range(NSLOT):
            drain_out(s)

    def wrapped(src, conn, mem, out, conn_l, r0, r1, r2, r3, t0, t1, t2, t3,
                o0, o1, o2, o3, zbuf, s0, s1, s2, s3, os0, os1, os2, os3):
        body(src, conn, mem, out, conn_l, (r0, r1, r2, r3), (t0, t1, t2, t3),
             (o0, o1, o2, o3), zbuf, (s0, s1, s2, s3), (os0, os1, os2, os3))

    return pl.kernel(
        wrapped,
        out_type=jax.ShapeDtypeStruct((out_rows, out_width), out_dtype),
        mesh=_MESH,
        compiler_params=pltpu.CompilerParams(needs_layout_passes=False),
        scratch_types=(
            [pltpu.VMEM((R, KP), jnp.int32)]
            + [pltpu.VMEM((KP, W), jnp.int32)] * NSLOT
            + [pltpu.VMEM((MEM,), jnp.float32)] * NSLOT
            + [pltpu.VMEM((out_width,), out_dtype)] * NSLOT
            + [pltpu.VMEM((16, W), jnp.int32)]
            + [pltpu.SemaphoreType.DMA] * (2 * NSLOT)
        ),
        name=f"ram_layer_{mode}",
    )


def _pad_conn(conn, r):
    n, k = conn.shape
    p = jnp.concatenate(
        [conn.astype(jnp.int32), jnp.zeros((n, KP - k), jnp.int32)], axis=1)
    return p.reshape(r, NW, KP).swapaxes(0, 1).reshape(n, KP)


@jax.jit
def kernel(x, conn_in, conn_state, conn_out, mem_in, mem_state, mem_out):
    xT = jax.lax.bitcast_convert_type(
        x.astype(jnp.bfloat16).T.reshape(INPUT_BITS, W, 2), jnp.int32)
    conn_in_p = _pad_conn(conn_in, 32)
    conn_state_p = _pad_conn(conn_state, 16)
    conn_out_p = _pad_conn(conn_out, 16)

    layer_in = _make_layer("input")
    layer_state = _make_layer("state")
    layer_out = _make_layer("out")

    bits = layer_in(xT, conn_in_p, mem_in)
    for _ in range(4):
        bits = layer_state(bits, conn_state_p, mem_state)
    out_t = layer_out(bits, conn_out_p, mem_out)
    out_t = out_t.reshape(N_OUT, NG, 2, 16).transpose(0, 1, 3, 2)
    return out_t.reshape(N_OUT, B).T


# speedup vs baseline: 2.3085x; 1.9258x over previous
import jax
import jax.numpy as jnp
from jax import lax
from jax.experimental import pallas as pl
from jax.experimental.pallas import tpu as pltpu
from jax.experimental.pallas import tpu_sc as plsc

NC, NS, LANES = 2, 16, 16
NW = NC * NS
NSLOT = 4

INPUT_BITS = 4096
N_IN = 1024
N_STATE = 512
N_OUT = 512
NBITS = N_IN + N_STATE
K = 14
KP = 16
MEM = 1 << K
B = 1024
W = B // 2
NG = B // 32

_FMT = plsc.PackFormat.INTERLEAVED

_MESH = plsc.VectorSubcoreMesh(
    core_axis_name="c", subcore_axis_name="s", num_cores=NC, num_subcores=NS
)


def _make_layer(mode):
    R = 32 if mode == "input" else 16
    out_rows = N_OUT if mode == "out" else NBITS
    out_width = B if mode == "out" else W
    out_dtype = jnp.float32 if mode == "out" else jnp.int32

    def body(src, conn, mem, out, conn_l, rows_, tab_, orow_, zbuf,
             sem_, osem_):
        wid = lax.axis_index("s") * NC + lax.axis_index("c")

        pltpu.sync_copy(conn.at[pl.ds(wid * R, R)], conn_l)

        def start(j, s):
            neuron = j * NW + wid
            pltpu.async_copy(src.at[conn_l.at[j, pl.ds(0, K)]],
                             rows_[s], sem_[s])
            pltpu.async_copy(mem.at[neuron], tab_[s], sem_[s])

        def wait_slot(s):
            pltpu.make_async_copy(src.at[conn_l.at[0, pl.ds(0, K)]],
                                  rows_[s], sem_[s]).wait()
            pltpu.make_async_copy(mem.at[0], tab_[s], sem_[s]).wait()

        def compute(j, s):
            rows16, tab, orow = rows_[s], tab_[s], orow_[s]
            neuron = j * NW + wid
            out_row = neuron + (N_IN if mode == "state" else 0)
            for g in range(NG):
                sl = pl.ds(g * 16, 16)
                r = [plsc.bitcast(rows16[k, sl], jnp.bfloat16)
                     for k in range(K)]
                lo = r[0]
                hi = r[7]
                for k in range(1, 7):
                    ck = jnp.bfloat16(1 << k)
                    lo = lo + r[k] * ck
                    hi = hi + r[k + 7] * ck
                lo_p, lo_q = plsc.unpack(lo, format=_FMT,
                                         preferred_element_type=jnp.float32)
                hi_p, hi_q = plsc.unpack(hi, format=_FMT,
                                         preferred_element_type=jnp.float32)
                bits = []
                for off, lo_h, hi_h in ((0, lo_p, hi_p), (16, lo_q, hi_q)):
                    a = (lo_h + hi_h * 128.0).astype(jnp.int32) & (MEM - 1)
                    v = plsc.load_gather(tab, [a])
                    if mode == "out":
                        orow[pl.ds(g * 32 + off, 16)] = v
                    else:
                        bits.append(jnp.where(v > 0.5, jnp.float32(1.0),
                                              jnp.float32(0.0)))
                if mode != "out":
                    orow[sl] = plsc.bitcast(
                        plsc.pack(bits[0], bits[1], format=_FMT), jnp.int32)
            pltpu.async_copy(orow, out.at[out_row], osem_[s])

        def drain_out(s):
            pltpu.make_async_copy(orow_[s], out.at[0], osem_[s]).wait()

        if mode == "input":
            for r in range(16):
                for g in range(W // 16):
                    zbuf[r, pl.ds(g * 16, 16)] = jnp.zeros((16,), jnp.int32)
            pltpu.sync_copy(zbuf, out.at[pl.ds(N_IN + wid * 16, 16)])
        elif mode == "state":
            for p in range(2):
                sl = pl.ds(wid * 32 + p * 16, 16)
                pltpu.sync_copy(src.at[sl], zbuf)
                pltpu.sync_copy(zbuf, out.at[sl])

        for i in range(NSLOT):
            start(i, i)

        @pl.loop(0, R, step=NSLOT)
        def _(j):
            for s in range(NSLOT):
                wait_slot(s)

                @pl.when(j + s >= NSLOT)
                def _():
                    drain_out(s)

                compute(j + s, s)

                @pl.when(j + s + NSLOT < R)
                def _():
                    start(j + s + NSLOT, s)

        for s in range(NSLOT):
            drain_out(s)

    def wrapped(src, conn, mem, out, conn_l, r0, r1, r2, r3, t0, t1, t2, t3,
                o0, o1, o2, o3, zbuf, s0, s1, s2, s3, os0, os1, os2, os3):
        body(src, conn, mem, out, conn_l, (r0, r1, r2, r3), (t0, t1, t2, t3),
             (o0, o1, o2, o3), zbuf, (s0, s1, s2, s3), (os0, os1, os2, os3))

    return pl.kernel(
        wrapped,
        out_type=jax.ShapeDtypeStruct((out_rows, out_width), out_dtype),
        mesh=_MESH,
        compiler_params=pltpu.CompilerParams(needs_layout_passes=False),
        scratch_types=(
            [pltpu.VMEM((R, KP), jnp.int32)]
            + [pltpu.VMEM((K, W), jnp.int32)] * NSLOT
            + [pltpu.VMEM((MEM,), jnp.float32)] * NSLOT
            + [pltpu.VMEM((out_width,), out_dtype)] * NSLOT
            + [pltpu.VMEM((16, W), jnp.int32)]
            + [pltpu.SemaphoreType.DMA] * (2 * NSLOT)
        ),
        name=f"ram_layer_{mode}",
    )


def _pad_conn(conn, r):
    n, k = conn.shape
    p = jnp.concatenate(
        [conn.astype(jnp.int32), jnp.zeros((n, KP - k), jnp.int32)], axis=1)
    return p.reshape(r, NW, KP).swapaxes(0, 1).reshape(n, KP)


@jax.jit
def kernel(x, conn_in, conn_state, conn_out, mem_in, mem_state, mem_out):
    xT = jax.lax.bitcast_convert_type(
        x.astype(jnp.bfloat16).T.reshape(INPUT_BITS, W, 2), jnp.int32)
    conn_in_p = _pad_conn(conn_in, 32)
    conn_state_p = _pad_conn(conn_state, 16)
    conn_out_p = _pad_conn(conn_out, 16)

    layer_in = _make_layer("input")
    layer_state = _make_layer("state")
    layer_out = _make_layer("out")

    bits = layer_in(xT, conn_in_p, mem_in)
    for _ in range(4):
        bits = layer_state(bits, conn_state_p, mem_state)
    out_t = layer_out(bits, conn_out_p, mem_out)
    out_t = out_t.reshape(N_OUT, NG, 2, 16).transpose(0, 1, 3, 2)
    return out_t.reshape(N_OUT, B).T
